# baseline (speedup 1.0000x reference)
import jax
import jax.numpy as jnp
from jax import lax
from jax.experimental import pallas as pl
from jax.experimental.pallas import tpu as pltpu
from jax.experimental.pallas import tpu_sc as plsc

B = 16384
Q = 50
QP = 56
D = 64
DP = 128
NC = 2
NS = 16
NW = NC * NS
PER_W = B // NW
NBUF = 8
LOOK = 4
N_GROUPS = PER_W // NBUF


def _emb_body(idx_hbm, table_hbm, out_hbm, idx_v, rows_v, *sems):
    gsems = sems[:NBUF]
    wsems = sems[NBUF:]
    wid = lax.axis_index("s") * NC + lax.axis_index("c")
    pltpu.sync_copy(idx_hbm.at[wid], idx_v)

    def fire_gather(k, b):
        pltpu.async_copy(
            table_hbm.at[idx_v.at[pl.ds(k * QP, QP)]],
            rows_v.at[b], gsems[b])

    def wait_gather(k, b):
        pltpu.make_async_copy(
            table_hbm.at[idx_v.at[pl.ds(k * QP, QP)]],
            rows_v.at[b], gsems[b]).wait()

    def fire_write(k, b):
        pltpu.async_copy(
            rows_v.at[b, pl.ds(0, Q), :],
            out_hbm.at[wid * PER_W + k], wsems[b])

    def wait_write(k, b):
        pltpu.make_async_copy(
            rows_v.at[b, pl.ds(0, Q), :],
            out_hbm.at[wid * PER_W + k], wsems[b]).wait()

    for b in range(LOOK):
        fire_gather(b, b)
    for j in range(NBUF):
        b = j % NBUF
        wait_gather(j, b)
        fire_write(j, b)
        b2 = (b + LOOK) % NBUF
        if j >= LOOK:
            wait_write(j - LOOK, b2)
        fire_gather(j + LOOK, b2)

    def group(g, carry):
        j0 = g * NBUF
        for b in range(NBUF):
            j = j0 + b
            wait_gather(j, b)
            fire_write(j, b)
            b2 = (b + LOOK) % NBUF
            wait_write(j - LOOK, b2)
            fire_gather(j + LOOK, b2)
        return carry

    lax.fori_loop(1, N_GROUPS - 1, group, 0)

    j0 = (N_GROUPS - 1) * NBUF
    for b in range(NBUF):
        j = j0 + b
        wait_gather(j, b)
        fire_write(j, b)
        b2 = (b + LOOK) % NBUF
        wait_write(j - LOOK, b2)
        if j + LOOK < PER_W:
            fire_gather(j + LOOK, b2)
    for b in range(NBUF - LOOK, NBUF):
        wait_write(j0 + b, b)


@jax.jit
def _emb_call(idx32, table):
    mesh = plsc.VectorSubcoreMesh(core_axis_name="c", subcore_axis_name="s")
    f = pl.kernel(
        _emb_body,
        out_type=jax.ShapeDtypeStruct((B, Q, D), jnp.float32),
        mesh=mesh,
        scratch_types=(
            [pltpu.VMEM((PER_W * QP,), jnp.int32),
             pltpu.VMEM((NBUF, QP, D), jnp.float32)]
            + [pltpu.SemaphoreType.DMA] * (2 * NBUF)
        ),
        compiler_params=pltpu.CompilerParams(use_tc_tiling_on_sc=False),
    )
    return f(idx32, table)


def kernel(idx, table):
    idx32 = jnp.pad(idx.astype(jnp.int32), ((0, 0), (0, QP - Q)))
    idx32 = idx32.reshape(NW, PER_W * QP)
    return _emb_call(idx32, table)
